# Initial kernel scaffold; baseline (speedup 1.0000x reference)
import functools
import jax
import jax.numpy as jnp
from jax import lax
from jax.experimental import pallas as pl
from jax.experimental.pallas import tpu as pltpu
from jax.experimental.pallas import tpu_sc as plsc

NW = 32
BLK = 64


def _sc_probe(P, dstc):
    E = dstc.shape[0]
    CH = E // NW
    NB = CH // BLK
    mesh = plsc.VectorSubcoreMesh(core_axis_name="c", subcore_axis_name="s")

    @functools.partial(
        pl.kernel,
        out_type=jax.ShapeDtypeStruct((NW * 320, 256), jnp.float32),
        mesh=mesh,
        scratch_types=[
            pltpu.VMEM((CH,), jnp.int32),
            pltpu.VMEM((BLK * 256,), jnp.float32),
            pltpu.VMEM((320 * 256,), jnp.float32),
            pltpu.SemaphoreType.DMA,
        ],
    )
    def k(P_hbm, dst_hbm, out_hbm, idx_v, rows_v, acc_v, sem):
        wid = lax.axis_index("s") * 2 + lax.axis_index("c")
        base = wid * CH
        pltpu.sync_copy(dst_hbm.at[pl.ds(base, CH)], idx_v)

        def init_body(i, _):
            acc_v[pl.ds(i * 16, 16)] = jnp.full((16,), -3.38953139e38, jnp.float32)
            return 0
        lax.fori_loop(0, 320 * 16, init_body, 0)

        lo = idx_v[0]
        lo = jnp.minimum(lo, 0)

        def blk_body(b, _):
            rows_2d = rows_v.reshape(BLK, 256)
            pltpu.async_copy(
                P_hbm.at[idx_v.at[pl.ds(b * BLK, BLK)]], rows_2d, sem
            ).wait()

            def edge_body(e, _):
                loc = idx_v[b * BLK + e]
                loc = jnp.minimum(loc, 319)

                def feat_body(j, _):
                    v = rows_v[pl.ds(e * 256 + j * 16, 16)]
                    v = jnp.exp(v * 1e-30)
                    a = acc_v[pl.ds(loc * 256 + j * 16, 16)]
                    acc_v[pl.ds(loc * 256 + j * 16, 16)] = jnp.maximum(a, v)
                    return 0
                lax.fori_loop(0, 16, feat_body, 0)
                return 0
            lax.fori_loop(lo, BLK, edge_body, 0)
            return 0
        lax.fori_loop(0, NB, blk_body, 0)

        iv = lax.iota(jnp.int32, 16)
        g = plsc.load_gather(acc_v, [iv])
        acc_v[pl.ds(0, 16)] = g

        acc_2d = acc_v.reshape(320, 256)
        pltpu.sync_copy(acc_2d, out_hbm.at[pl.ds(wid * 320, 320)])

    return k(P, dstc)


def kernel(x, edge_index, edge_attr, ne_w1, ne_b1, ne_w2, ne_b2, ee_w1, ee_b1,
           ee_w2, ee_b2, conv_w1, conv_b1, conv_w2, conv_b2, gat_w,
           gat_att_src, gat_att_dst, gat_bias, off_w1, off_b1, off_w2, off_b2,
           lat_w1, lat_b1, lat_w2, lat_b2, en_w1, en_b1, en_w2, en_b2):
    dst = edge_index[1]
    dstc = jnp.sort(dst)
    o = _sc_probe(x, dstc)
    s = jnp.sum(o[:1, :1])
    z = jnp.zeros((x.shape[0], 1), jnp.float32) + s
    return (z, z, z)


# speedup vs baseline: 2.1356x; 2.1356x over previous
import functools

import jax
import jax.numpy as jnp
from jax import lax
from jax.experimental import pallas as pl
from jax.experimental.pallas import tpu as pltpu
from jax.experimental.pallas import tpu_sc as plsc

N = 10000
H = 256
NP = 10240
RPT = 320
NW = 32
E = 160000
EP_C = 163840
EG = E + N
EP_G = 170496
BLK = 64
NEG = -3.38953139e38

_mesh = plsc.VectorSubcoreMesh(core_axis_name="c", subcore_axis_name="s")
_sc_params = pltpu.CompilerParams(needs_layout_passes=False)


def _tc_node_mlp(xp, w1, b1r, w2, b2r):
    def body(x_ref, w1_ref, b1_ref, w2_ref, b2_ref, o_ref):
        t = jnp.maximum(
            jnp.dot(x_ref[...], w1_ref[...], preferred_element_type=jnp.float32) + b1_ref[...], 0.0)
        o_ref[...] = jnp.dot(t, w2_ref[...], preferred_element_type=jnp.float32) + b2_ref[...]

    return pl.pallas_call(
        body,
        grid=(NP // 256,),
        in_specs=[
            pl.BlockSpec((256, H), lambda i: (i, 0)),
            pl.BlockSpec((H, H), lambda i: (0, 0)),
            pl.BlockSpec((1, H), lambda i: (0, 0)),
            pl.BlockSpec((H, H), lambda i: (0, 0)),
            pl.BlockSpec((1, H), lambda i: (0, 0)),
        ],
        out_specs=pl.BlockSpec((256, H), lambda i: (i, 0)),
        out_shape=jax.ShapeDtypeStruct((NP, H), jnp.float32),
    )(xp, w1, b1r, w2, b2r)


def _tc_layer_mats(h, xc, xa, gbr, wcat, bcat, avs_r, avd_r, combine):
    def body(*refs):
        if combine:
            (h_ref, xc_ref, xa_ref, gb_ref, wcat_ref, bcat_ref, avs_ref,
             avd_ref, hn_ref, p_ref, q_ref, xw_ref, a2_ref) = refs
            hv = jnp.maximum(
                h_ref[...] + xc_ref[...] + xa_ref[...] + gb_ref[...], 0.0)
        else:
            (h_ref, wcat_ref, bcat_ref, avs_ref, avd_ref, hn_ref, p_ref,
             q_ref, xw_ref, a2_ref) = refs
            hv = h_ref[...]
        hn_ref[...] = hv
        pqx = jnp.dot(hv, wcat_ref[...], preferred_element_type=jnp.float32) + bcat_ref[...]
        p_ref[...] = pqx[:, 0:H]
        q_ref[...] = pqx[:, H:2 * H]
        xw = pqx[:, 2 * H:3 * H]
        xw_ref[...] = xw
        dn = (((1,), (1,)), ((), ()))
        a_s = lax.dot_general(avs_ref[...], xw, dn, preferred_element_type=jnp.float32)
        a_d = lax.dot_general(avd_ref[...], xw, dn, preferred_element_type=jnp.float32)
        a2_ref[...] = jnp.concatenate(
            [a_s, a_d, jnp.zeros((14, 128), jnp.float32)], axis=0)

    node_spec = pl.BlockSpec((128, H), lambda i: (i, 0))
    full_spec = lambda shape: pl.BlockSpec(shape, lambda i: (0, 0))
    in_specs = [node_spec]
    args = [h]
    if combine:
        in_specs += [node_spec, node_spec, full_spec((1, H))]
        args += [xc, xa, gbr]
    in_specs += [full_spec((H, 3 * H)), full_spec((1, 3 * H)),
                 full_spec((1, H)), full_spec((1, H))]
    args += [wcat, bcat, avs_r, avd_r]

    return pl.pallas_call(
        body,
        grid=(NP // 128,),
        in_specs=in_specs,
        out_specs=[node_spec, node_spec, node_spec, node_spec,
                   pl.BlockSpec((16, 128), lambda i: (0, i))],
        out_shape=[
            jax.ShapeDtypeStruct((NP, H), jnp.float32),
            jax.ShapeDtypeStruct((NP, H), jnp.float32),
            jax.ShapeDtypeStruct((NP, H), jnp.float32),
            jax.ShapeDtypeStruct((NP, H), jnp.float32),
            jax.ShapeDtypeStruct((16, NP), jnp.float32),
        ],
    )(*args)


def _tc_msg_mlp(pre, w2, b2r):
    def body(pre_ref, w2_ref, b2_ref, o_ref):
        t = jnp.maximum(pre_ref[...], 0.0)
        o_ref[...] = jnp.dot(t, w2_ref[...], preferred_element_type=jnp.float32) + b2_ref[...]

    return pl.pallas_call(
        body,
        grid=(EP_C // 256,),
        in_specs=[
            pl.BlockSpec((256, H), lambda i: (i, 0)),
            pl.BlockSpec((H, H), lambda i: (0, 0)),
            pl.BlockSpec((1, H), lambda i: (0, 0)),
        ],
        out_specs=pl.BlockSpec((256, H), lambda i: (i, 0)),
        out_shape=jax.ShapeDtypeStruct((EP_C, H), jnp.float32),
    )(pre, w2, b2r)


def _tc_heads(h, xc, xa, gbr, w1c, b1c, w2b, b2r):
    def body(h_ref, xc_ref, xa_ref, gb_ref, w1_ref, b1_ref, w2_ref, b2_ref,
             o_ref):
        hv = jnp.maximum(
            h_ref[...] + xc_ref[...] + xa_ref[...] + gb_ref[...], 0.0)
        t = jnp.maximum(
            jnp.dot(hv, w1_ref[...], preferred_element_type=jnp.float32) + b1_ref[...], 0.0)
        o_ref[...] = jnp.dot(t, w2_ref[...], preferred_element_type=jnp.float32) + b2_ref[...]

    node_spec = pl.BlockSpec((128, H), lambda i: (i, 0))
    full_spec = lambda shape: pl.BlockSpec(shape, lambda i: (0, 0))
    return pl.pallas_call(
        body,
        grid=(NP // 128,),
        in_specs=[node_spec, node_spec, node_spec, full_spec((1, H)),
                  full_spec((H, 3 * H)), full_spec((1, 3 * H)),
                  full_spec((3 * H, 128)), full_spec((1, 128))],
        out_specs=pl.BlockSpec((128, 128), lambda i: (i, 0)),
        out_shape=jax.ShapeDtypeStruct((NP, 128), jnp.float32),
    )(h, xc, xa, gbr, w1c, b1c, w2b, b2r)


def _sc_gather_pre(P, Q, dst_g, src_g):
    CH = EP_C // NW
    NB = CH // BLK

    @functools.partial(
        pl.kernel,
        out_type=jax.ShapeDtypeStruct((EP_C, H), jnp.float32),
        mesh=_mesh,
        scratch_types=[
            pltpu.VMEM((CH,), jnp.int32),
            pltpu.VMEM((CH,), jnp.int32),
            pltpu.VMEM((BLK, H), jnp.float32),
            pltpu.VMEM((BLK, H), jnp.float32),
            pltpu.SemaphoreType.DMA,
            pltpu.SemaphoreType.DMA,
        ],
        compiler_params=_sc_params,
    )
    def k(p_h, q_h, d_h, s_h, out_h, didx, sidx, prow, qrow, sem1, sem2):
        wid = lax.axis_index("s") * 2 + lax.axis_index("c")
        base = wid * CH
        pltpu.sync_copy(d_h.at[pl.ds(base, CH)], didx)
        pltpu.sync_copy(s_h.at[pl.ds(base, CH)], sidx)

        def blk_body(b, _):
            cp1 = pltpu.async_copy(
                p_h.at[didx.at[pl.ds(b * BLK, BLK)]], prow, sem1)
            cp2 = pltpu.async_copy(
                q_h.at[sidx.at[pl.ds(b * BLK, BLK)]], qrow, sem2)
            cp1.wait()
            cp2.wait()

            def e_body(e, _):
                def f_body(j, _):
                    prow[e, pl.ds(j * 16, 16)] = (
                        prow[e, pl.ds(j * 16, 16)]
                        + qrow[e, pl.ds(j * 16, 16)])
                    return 0
                lax.fori_loop(0, 16, f_body, 0)
                return 0
            lax.fori_loop(0, BLK, e_body, 0)
            pltpu.sync_copy(prow, out_h.at[pl.ds(base + b * BLK, BLK)])
            return 0
        lax.fori_loop(0, NB, blk_body, 0)

    return k(P, Q, dst_g, src_g)


def _sc_reduce(M, dstc, bc, srcg, dstg, bg, A2, XW):
    @functools.partial(
        pl.kernel,
        out_type=[jax.ShapeDtypeStruct((NP, H), jnp.float32),
                  jax.ShapeDtypeStruct((NP, H), jnp.float32)],
        mesh=_mesh,
        scratch_types=[
            pltpu.VMEM((328, H), jnp.float32),
            pltpu.VMEM((BLK, H), jnp.float32),
            pltpu.VMEM((80,), jnp.int32),
            pltpu.VMEM((80,), jnp.int32),
            pltpu.VMEM((1, NP), jnp.float32),
            pltpu.VMEM((1, NP), jnp.float32),
            pltpu.VMEM((5632,), jnp.float32),
            pltpu.VMEM((352,), jnp.float32),
            pltpu.VMEM((64,), jnp.int32),
            pltpu.VMEM((32,), jnp.int32),
            pltpu.VMEM((32,), jnp.int32),
            pltpu.VMEM((32,), jnp.float32),
            pltpu.SemaphoreType.DMA,
        ],
        compiler_params=_sc_params,
    )
    def k(m_h, dc_h, bc_h, sg_h, dg_h, bg_h, a2_h, xw_h, xc_h, xa_h,
          acc, mrow, dblk, sblk, asrc, adst, emx, dnm, bcv, locb, loc2b,
          valb, sem):
        wid = lax.axis_index("s") * 2 + lax.axis_index("c")
        base = wid * RPT
        iota16 = lax.iota(jnp.int32, 16)
        zero16 = jnp.zeros((16,), jnp.int32)
        negv = jnp.full((16,), NEG, jnp.float32)

        pltpu.sync_copy(bc_h.at[pl.ds(0, 48)], bcv.at[pl.ds(0, 48)])
        lo = bcv[pl.ds(wid, 16)][0]
        hi = bcv[pl.ds(wid + 1, 16)][0]

        def initA(i, _):
            def initf(j, _):
                acc[i, pl.ds(j * 16, 16)] = negv
                return 0
            lax.fori_loop(0, 16, initf, 0)
            return 0
        lax.fori_loop(0, 328, initA, 0)

        lo64 = pl.multiple_of(lo - lax.rem(lo, 64), 64)
        nb = (hi - lo64 + 63) // 64

        def blkA(b, _):
            s = lo64 + b * BLK
            pltpu.sync_copy(dc_h.at[pl.ds(s, BLK)], dblk.at[pl.ds(0, BLK)])
            pltpu.sync_copy(m_h.at[pl.ds(s, BLK)], mrow)

            def eA(e, _):
                d = dblk[pl.ds(e, 16)][0]
                g = s + e
                valid = (g >= lo) & (g < hi)
                loc = jnp.where(valid, d - base, 320)

                def fA(j, _):
                    a = acc[loc, pl.ds(j * 16, 16)]
                    v = mrow[e, pl.ds(j * 16, 16)]
                    acc[loc, pl.ds(j * 16, 16)] = jnp.maximum(a, v)
                    return 0
                lax.fori_loop(0, 16, fA, 0)
                return 0
            lax.fori_loop(0, BLK, eA, 0)
            return 0
        lax.fori_loop(0, nb, blkA, 0)

        def finA(i, _):
            def finf(j, _):
                a = acc[i, pl.ds(j * 16, 16)]
                acc[i, pl.ds(j * 16, 16)] = jnp.where(a == negv, 0.0, a)
                return 0
            lax.fori_loop(0, 16, finf, 0)
            return 0
        lax.fori_loop(0, RPT, finA, 0)
        pltpu.sync_copy(acc.at[pl.ds(0, RPT)], xc_h.at[pl.ds(base, RPT)])

        pltpu.sync_copy(a2_h.at[pl.ds(0, 1)], asrc)
        pltpu.sync_copy(a2_h.at[pl.ds(1, 1)], adst)
        pltpu.sync_copy(bg_h.at[pl.ds(0, 48)], bcv.at[pl.ds(0, 48)])
        lo2 = bcv[pl.ds(wid, 16)][0]
        hi2 = bcv[pl.ds(wid + 1, 16)][0]

        def initB(i, _):
            emx[pl.ds(i * 16, 16)] = negv
            return 0
        lax.fori_loop(0, 352, initB, 0)

        def initB2(i, _):
            dnm[pl.ds(i * 16, 16)] = jnp.zeros((16,), jnp.float32)
            return 0
        lax.fori_loop(0, 22, initB2, 0)

        lo64b = pl.multiple_of(lo2 - lax.rem(lo2, 64), 64)
        nb2 = (hi2 - lo64b + 63) // 64

        def _edge_vals(s, kk):
            sv = sblk[pl.ds(kk * 16, 16)]
            dv = dblk[pl.ds(kk * 16, 16)]
            asv = plsc.load_gather(asrc, [zero16, sv])
            adv = plsc.load_gather(adst, [zero16, dv])
            ev = asv + adv
            ev = jnp.where(ev > 0, ev, 0.2 * ev)
            gv = s + kk * 16 + iota16
            validv = (gv >= lo2) & (gv < hi2)
            return dv, ev, validv

        def blkB1(b, _):
            s = lo64b + b * BLK
            pltpu.sync_copy(dg_h.at[pl.ds(s, BLK)], dblk.at[pl.ds(0, BLK)])
            pltpu.sync_copy(sg_h.at[pl.ds(s, BLK)], sblk.at[pl.ds(0, BLK)])

            def grp(kk, _):
                dv, ev, validv = _edge_vals(s, kk)
                locv = jnp.where(validv, dv - base,
                                 jnp.full((16,), 336, jnp.int32))
                loc2b[pl.ds(0, 16)] = locv
                valb[pl.ds(0, 16)] = ev

                def lane(i, _):
                    loc = loc2b[pl.ds(i, 16)][0]
                    val = valb[pl.ds(i, 16)][0]
                    off = pl.multiple_of(loc * 16, 16)
                    cur = emx[pl.ds(off, 16)]
                    emx[pl.ds(off, 16)] = jnp.maximum(
                        cur, jnp.full((16,), val))
                    return 0
                lax.fori_loop(0, 16, lane, 0)
                return 0
            lax.fori_loop(0, 4, grp, 0)
            return 0
        lax.fori_loop(0, nb2, blkB1, 0)

        def blkB2(b, _):
            s = lo64b + b * BLK
            pltpu.sync_copy(dg_h.at[pl.ds(s, BLK)], dblk.at[pl.ds(0, BLK)])
            pltpu.sync_copy(sg_h.at[pl.ds(s, BLK)], sblk.at[pl.ds(0, BLK)])

            def grp(kk, _):
                dv, ev, validv = _edge_vals(s, kk)
                locv = jnp.where(validv, dv - base,
                                 jnp.full((16,), 336, jnp.int32))
                mg = plsc.load_gather(emx, [locv * 16])
                exv = jnp.exp(ev - mg)
                plsc.addupdate_scatter(dnm, [locv], exv)
                return 0
            lax.fori_loop(0, 4, grp, 0)
            return 0
        lax.fori_loop(0, nb2, blkB2, 0)

        def initA2(i, _):
            def initf(j, _):
                acc[i, pl.ds(j * 16, 16)] = jnp.zeros((16,), jnp.float32)
                return 0
            lax.fori_loop(0, 16, initf, 0)
            return 0
        lax.fori_loop(0, 328, initA2, 0)

        def blkB3(b, _):
            s = lo64b + b * BLK
            pltpu.sync_copy(dg_h.at[pl.ds(s, BLK)], dblk.at[pl.ds(0, BLK)])
            pltpu.sync_copy(sg_h.at[pl.ds(s, BLK)], sblk.at[pl.ds(0, BLK)])
            pltpu.async_copy(
                xw_h.at[sblk.at[pl.ds(0, BLK)]], mrow, sem).wait()

            def grp(kk, _):
                dv, ev, validv = _edge_vals(s, kk)
                locdv = jnp.where(validv, dv - base,
                                  jnp.full((16,), 336, jnp.int32))
                mg = plsc.load_gather(emx, [locdv * 16])
                dg = plsc.load_gather(dnm, [locdv])
                exv = jnp.exp(ev - mg)
                alphav = exv / (dg + 1e-16)
                alphav = jnp.where(validv, alphav, 0.0)
                locv = jnp.where(validv, dv - base,
                                 jnp.full((16,), 320, jnp.int32))
                locb[pl.ds(0, 16)] = locv
                valb[pl.ds(0, 16)] = alphav

                def lane(i, _):
                    loc = locb[pl.ds(i, 16)][0]
                    al = valb[pl.ds(i, 16)][0]
                    alb = jnp.full((16,), al)
                    er = kk * 16 + i

                    locr = jnp.full((16,), loc, jnp.int32)

                    def fB(j, _):
                        plsc.addupdate_scatter(
                            acc, [locr, j * 16 + iota16],
                            alb * mrow[er, pl.ds(j * 16, 16)])
                        return 0
                    lax.fori_loop(0, 16, fB, 0)
                    return 0
                lax.fori_loop(0, 16, lane, 0)
                return 0
            lax.fori_loop(0, 4, grp, 0)
            return 0
        lax.fori_loop(0, nb2, blkB3, 0)

        pltpu.sync_copy(acc.at[pl.ds(0, RPT)], xa_h.at[pl.ds(base, RPT)])

    return k(M, dstc, bc, srcg, dstg, bg, A2, XW)


def kernel(x, edge_index, edge_attr, ne_w1, ne_b1, ne_w2, ne_b2, ee_w1, ee_b1,
           ee_w2, ee_b2, conv_w1, conv_b1, conv_w2, conv_b2, gat_w,
           gat_att_src, gat_att_dst, gat_bias, off_w1, off_b1, off_w2, off_b2,
           lat_w1, lat_b1, lat_w2, lat_b2, en_w1, en_b1, en_w2, en_b2):
    i32 = jnp.int32
    src = edge_index[0].astype(i32)
    dst = edge_index[1].astype(i32)

    perm = jnp.argsort(dst)
    dstc = dst[perm]
    srcc = src[perm]
    sentinel_c = jnp.full((EP_C - E,), NP, i32)
    dstc_srt = jnp.concatenate([dstc, sentinel_c])
    dstc_gth = jnp.minimum(dstc_srt, N - 1)
    srcc_gth = jnp.concatenate([srcc, jnp.zeros((EP_C - E,), i32)])
    bounds = jnp.arange(33, dtype=i32) * RPT
    bc = jnp.searchsorted(dstc_srt, bounds).astype(i32)
    bc = jnp.concatenate([bc, jnp.zeros((15,), i32)])

    self_idx = jnp.arange(N, dtype=i32)
    srcl = jnp.concatenate([src, self_idx])
    dstl = jnp.concatenate([dst, self_idx])
    permg = jnp.argsort(dstl)
    dstg = dstl[permg]
    srcg = srcl[permg]
    dstg_p = jnp.concatenate([dstg, jnp.full((EP_G - EG,), NP, i32)])
    srcg_p = jnp.concatenate([srcg, jnp.zeros((EP_G - EG,), i32)])
    bg = jnp.searchsorted(dstg_p, bounds).astype(i32)
    bg = jnp.concatenate([bg, jnp.zeros((15,), i32)])

    xp = jnp.concatenate(
        [x, jnp.zeros((NP - N, x.shape[1]), jnp.float32)], axis=0)

    h = _tc_node_mlp(xp, ne_w1, ne_b1.reshape(1, -1), ne_w2,
                     ne_b2.reshape(1, -1))

    xc = xa = None
    for l in range(conv_w1.shape[0]):
        w1a = conv_w1[l][:H]
        w1b = conv_w1[l][H:]
        wcat = jnp.concatenate([w1a, w1b, gat_w[l]], axis=1)
        bcat = jnp.concatenate(
            [conv_b1[l], jnp.zeros((2 * H,), jnp.float32)]).reshape(1, -1)
        avs_r = gat_att_src[l].reshape(1, -1)
        avd_r = gat_att_dst[l].reshape(1, -1)
        if l == 0:
            h, P, Q, XW, A2 = _tc_layer_mats(
                h, None, None, None, wcat, bcat, avs_r, avd_r, False)
        else:
            h, P, Q, XW, A2 = _tc_layer_mats(
                h, xc, xa, gat_bias[l - 1].reshape(1, -1), wcat, bcat,
                avs_r, avd_r, True)
        pre = _sc_gather_pre(P, Q, dstc_gth, srcc_gth)
        M = _tc_msg_mlp(pre, conv_w2[l], conv_b2[l].reshape(1, -1))
        xc, xa = _sc_reduce(M, dstc_srt, bc, srcg_p, dstg_p, bg, A2, XW)

    w1c = jnp.concatenate([off_w1, lat_w1, en_w1], axis=1)
    b1c = jnp.concatenate([off_b1, lat_b1, en_b1]).reshape(1, -1)
    w2b = jnp.zeros((3 * H, 128), jnp.float32)
    w2b = w2b.at[0:H, 0].set(off_w2[:, 0])
    w2b = w2b.at[H:2 * H, 1].set(lat_w2[:, 0])
    w2b = w2b.at[2 * H:3 * H, 2].set(en_w2[:, 0])
    b2r = jnp.zeros((1, 128), jnp.float32)
    b2r = b2r.at[0, 0].set(off_b2[0])
    b2r = b2r.at[0, 1].set(lat_b2[0])
    b2r = b2r.at[0, 2].set(en_b2[0])

    out = _tc_heads(h, xc, xa, gat_bias[2].reshape(1, -1), w1c, b1c, w2b, b2r)
    return (out[:N, 0:1], out[:N, 1:2], out[:N, 2:3])
